# Initial kernel scaffold; baseline (speedup 1.0000x reference)
import functools

import jax
import jax.numpy as jnp
from jax import lax
from jax.experimental import pallas as pl
from jax.experimental.pallas import tpu as pltpu
from jax.experimental.pallas import tpu_sc as plsc

NC = 2
NS = 16
NW = NC * NS

CH = 100


def _sc_degree(dst3, z16, ones16, n):
    nch = dst3.shape[1]
    rows = n // NS
    mesh = plsc.VectorSubcoreMesh(core_axis_name="c", subcore_axis_name="s")

    @functools.partial(
        pl.kernel,
        out_type=jax.ShapeDtypeStruct((NC, n, 16), jnp.float32),
        mesh=mesh,
        scratch_types=[
            pltpu.VMEM((nch, CH), jnp.int32),
            pltpu.VMEM((CH, 16), jnp.float32),
            pltpu.VMEM_SHARED((n, 16), jnp.float32),
        ],
    )
    def k(dst_hbm, z_hbm, ones_hbm, out_hbm, didx, ones_v, acc):
        c = lax.axis_index("c")
        s = lax.axis_index("s")
        wid = s * NC + c
        pltpu.sync_copy(z_hbm.at[pl.ds(s * rows, rows)], acc.at[pl.ds(s * rows, rows)])
        pltpu.sync_copy(ones_hbm, ones_v)
        pltpu.sync_copy(dst_hbm.at[wid], didx)
        plsc.subcore_barrier()

        def body(j, carry):
            pltpu.sync_copy(ones_v, acc.at[didx.at[j]], add=True)
            return carry

        lax.fori_loop(0, nch, body, 0)
        plsc.subcore_barrier()
        pltpu.sync_copy(acc.at[pl.ds(s * rows, rows)], out_hbm.at[c, pl.ds(s * rows, rows)])

    return k(dst3, z16, ones16)


def _sc_aggregate(g, src3, dst3, zeros, n, d):
    nch = src3.shape[1]
    rows = n // NS
    mesh = plsc.VectorSubcoreMesh(core_axis_name="c", subcore_axis_name="s")

    @functools.partial(
        pl.kernel,
        out_type=jax.ShapeDtypeStruct((NC, n, d), jnp.float32),
        mesh=mesh,
        scratch_types=[
            pltpu.VMEM((nch, CH), jnp.int32),
            pltpu.VMEM((nch, CH), jnp.int32),
            pltpu.VMEM((CH, d), jnp.float32),
            pltpu.VMEM_SHARED((n, d), jnp.float32),
            pltpu.SemaphoreType.DMA,
        ],
    )
    def k(g_hbm, src_hbm, dst_hbm, z_hbm, out_hbm, sidx, didx, buf, acc, sem):
        c = lax.axis_index("c")
        s = lax.axis_index("s")
        wid = s * NC + c
        pltpu.sync_copy(z_hbm.at[pl.ds(s * rows, rows)], acc.at[pl.ds(s * rows, rows)])
        pltpu.sync_copy(src_hbm.at[wid], sidx)
        pltpu.sync_copy(dst_hbm.at[wid], didx)
        plsc.subcore_barrier()

        def body(j, carry):
            pltpu.async_copy(g_hbm.at[sidx.at[j]], buf, sem).wait()
            pltpu.sync_copy(buf, acc.at[didx.at[j]], add=True)
            return carry

        lax.fori_loop(0, nch, body, 0)
        plsc.subcore_barrier()
        pltpu.sync_copy(acc.at[pl.ds(s * rows, rows)], out_hbm.at[c, pl.ds(s * rows, rows)])

    return k(g, src3, dst3, zeros)


def _tc_scale(degp, x, bn):
    n, d = x.shape
    grid = n // bn

    def body(degp_ref, x_ref, g0_ref, dinv_ref):
        deg = degp_ref[0][:, 0:1] + degp_ref[1][:, 0:1] + 1.0
        dinv = lax.rsqrt(deg)
        dinv_ref[...] = dinv
        g0_ref[...] = x_ref[...] * dinv

    return pl.pallas_call(
        body,
        grid=(grid,),
        in_specs=[
            pl.BlockSpec((NC, bn, 16), lambda i: (0, i, 0)),
            pl.BlockSpec((bn, d), lambda i: (i, 0)),
        ],
        out_specs=[
            pl.BlockSpec((bn, d), lambda i: (i, 0)),
            pl.BlockSpec((bn, 1), lambda i: (i, 0)),
        ],
        out_shape=[
            jax.ShapeDtypeStruct((n, d), jnp.float32),
            jax.ShapeDtypeStruct((n, 1), jnp.float32),
        ],
    )(degp, x)


def _tc_mid(p, x, dinv, W1, b1, W2, bn):
    n, d = x.shape
    dh = W1.shape[1]
    do = W2.shape[1]
    grid = n // bn

    def body(p_ref, x_ref, dinv_ref, W1_ref, b1_ref, W2_ref, t_ref, g1_ref):
        dinv = dinv_ref[...]
        sx = dinv * (p_ref[0] + p_ref[1]) + (dinv * dinv) * x_ref[...]
        h = jnp.dot(sx, W1_ref[...], preferred_element_type=jnp.float32)
        h = jnp.maximum(h + b1_ref[...], 0.0)
        t = jnp.dot(h, W2_ref[...], preferred_element_type=jnp.float32)
        t_ref[...] = t
        g1_ref[...] = dinv * t

    return pl.pallas_call(
        body,
        grid=(grid,),
        in_specs=[
            pl.BlockSpec((NC, bn, d), lambda i: (0, i, 0)),
            pl.BlockSpec((bn, d), lambda i: (i, 0)),
            pl.BlockSpec((bn, 1), lambda i: (i, 0)),
            pl.BlockSpec((d, dh), lambda i: (0, 0)),
            pl.BlockSpec((1, dh), lambda i: (0, 0)),
            pl.BlockSpec((dh, do), lambda i: (0, 0)),
        ],
        out_specs=[
            pl.BlockSpec((bn, do), lambda i: (i, 0)),
            pl.BlockSpec((bn, do), lambda i: (i, 0)),
        ],
        out_shape=[
            jax.ShapeDtypeStruct((n, do), jnp.float32),
            jax.ShapeDtypeStruct((n, do), jnp.float32),
        ],
    )(p, x, dinv, W1, b1, W2)


def _tc_final(q, t, dinv, b2, bn):
    n, do = t.shape
    grid = n // bn

    def body(q_ref, t_ref, dinv_ref, b2_ref, o_ref):
        dinv = dinv_ref[...]
        o_ref[...] = (dinv * (q_ref[0] + q_ref[1])
                      + (dinv * dinv) * t_ref[...] + b2_ref[...])

    return pl.pallas_call(
        body,
        grid=(grid,),
        in_specs=[
            pl.BlockSpec((NC, bn, do), lambda i: (0, i, 0)),
            pl.BlockSpec((bn, do), lambda i: (i, 0)),
            pl.BlockSpec((bn, 1), lambda i: (i, 0)),
            pl.BlockSpec((1, do), lambda i: (0, 0)),
        ],
        out_specs=pl.BlockSpec((bn, do), lambda i: (i, 0)),
        out_shape=jax.ShapeDtypeStruct((n, do), jnp.float32),
    )(q, t, dinv, b2)


def kernel(x, edge_index, W1, b1, W2, b2):
    n, d_in = x.shape
    e = edge_index.shape[1]
    ept = e // NW
    nch = ept // CH
    bn = 1000

    src3 = edge_index[0].reshape(NW, nch, CH)
    dst3 = edge_index[1].reshape(NW, nch, CH)
    zeros = jnp.zeros((n, d_in), jnp.float32)
    z16 = jnp.zeros((n, 16), jnp.float32)
    ones16 = jnp.ones((CH, 16), jnp.float32)

    degp = _sc_degree(dst3, z16, ones16, n)
    g0, dinv = _tc_scale(degp, x, bn)
    p1 = _sc_aggregate(g0, src3, dst3, zeros, n, d_in)
    t, g1 = _tc_mid(p1, x, dinv, W1, b1.reshape(1, -1), W2, bn)
    p2 = _sc_aggregate(g1, src3, dst3, zeros, n, d_in)
    return _tc_final(p2, t, dinv, b2.reshape(1, -1), bn)


# speedup vs baseline: 21.5454x; 21.5454x over previous
import functools

import jax
import jax.numpy as jnp
from jax import lax
from jax.experimental import pallas as pl
from jax.experimental.pallas import tpu as pltpu
from jax.experimental.pallas import tpu_sc as plsc

NC = 2
NS = 16
NW = NC * NS

CH = 100


def _sc_degree(dst3, zeros, ones128, n):
    nch = dst3.shape[1]
    np_, d = zeros.shape
    rows = np_ // NS
    mesh = plsc.VectorSubcoreMesh(core_axis_name="c", subcore_axis_name="s")

    @functools.partial(
        pl.kernel,
        out_type=jax.ShapeDtypeStruct((NC, np_, d), jnp.float32),
        mesh=mesh,
        scratch_types=[
            pltpu.VMEM((nch, CH), jnp.int32),
            pltpu.VMEM((CH, d), jnp.float32),
            pltpu.VMEM_SHARED((np_, d), jnp.float32),
        ],
    )
    def k(dst_hbm, z_hbm, ones_hbm, out_hbm, didx, ones_v, acc):
        c = lax.axis_index("c")
        s = lax.axis_index("s")
        wid = s * NC + c
        pltpu.sync_copy(z_hbm.at[pl.ds(s * rows, rows)], acc.at[pl.ds(s * rows, rows)])
        pltpu.sync_copy(ones_hbm, ones_v)
        pltpu.sync_copy(dst_hbm.at[wid], didx)
        plsc.subcore_barrier()

        def body(j, carry):
            pltpu.sync_copy(ones_v, acc.at[didx.at[j]], add=True)
            return carry

        lax.fori_loop(0, nch, body, 0)
        plsc.subcore_barrier()
        pltpu.sync_copy(acc.at[pl.ds(s * rows, rows)], out_hbm.at[c, pl.ds(s * rows, rows)])

    return k(dst3, zeros, ones128)


def _sc_aggregate(g, src3, dst3, zeros, n, d):
    nch = src3.shape[1]
    np_ = zeros.shape[0]
    rows = np_ // NS
    mesh = plsc.VectorSubcoreMesh(core_axis_name="c", subcore_axis_name="s")

    @functools.partial(
        pl.kernel,
        out_type=jax.ShapeDtypeStruct((NC, np_, d), jnp.float32),
        mesh=mesh,
        scratch_types=[
            pltpu.VMEM((nch, CH), jnp.int32),
            pltpu.VMEM((nch, CH), jnp.int32),
            pltpu.VMEM((CH, d), jnp.float32),
            pltpu.VMEM_SHARED((np_, d), jnp.float32),
            pltpu.SemaphoreType.DMA,
        ],
    )
    def k(g_hbm, src_hbm, dst_hbm, z_hbm, out_hbm, sidx, didx, buf, acc, sem):
        c = lax.axis_index("c")
        s = lax.axis_index("s")
        wid = s * NC + c
        pltpu.sync_copy(z_hbm.at[pl.ds(s * rows, rows)], acc.at[pl.ds(s * rows, rows)])
        pltpu.sync_copy(src_hbm.at[wid], sidx)
        pltpu.sync_copy(dst_hbm.at[wid], didx)
        plsc.subcore_barrier()

        def body(j, carry):
            pltpu.async_copy(g_hbm.at[sidx.at[j]], buf, sem).wait()
            pltpu.sync_copy(buf, acc.at[didx.at[j]], add=True)
            return carry

        lax.fori_loop(0, nch, body, 0)
        plsc.subcore_barrier()
        pltpu.sync_copy(acc.at[pl.ds(s * rows, rows)], out_hbm.at[c, pl.ds(s * rows, rows)])

    return k(g, src3, dst3, zeros)


def _tc_scale(degp, x, bn):
    n, d = x.shape
    grid = n // bn

    def body(degp_ref, x_ref, g0_ref, dinv_ref):
        deg = degp_ref[0][:, 0:1] + degp_ref[1][:, 0:1] + 1.0
        dinv = lax.rsqrt(deg)
        dinv_ref[...] = dinv
        g0_ref[...] = x_ref[...] * dinv

    return pl.pallas_call(
        body,
        grid=(grid,),
        in_specs=[
            pl.BlockSpec((NC, bn, d), lambda i: (0, i, 0)),
            pl.BlockSpec((bn, d), lambda i: (i, 0)),
        ],
        out_specs=[
            pl.BlockSpec((bn, d), lambda i: (i, 0)),
            pl.BlockSpec((bn, 1), lambda i: (i, 0)),
        ],
        out_shape=[
            jax.ShapeDtypeStruct((n, d), jnp.float32),
            jax.ShapeDtypeStruct((n, 1), jnp.float32),
        ],
    )(degp, x)


def _tc_mid(p, x, dinv, W1, b1, W2, bn):
    n, d = x.shape
    dh = W1.shape[1]
    do = W2.shape[1]
    grid = n // bn

    def body(p_ref, x_ref, dinv_ref, W1_ref, b1_ref, W2_ref, t_ref, g1_ref):
        dinv = dinv_ref[...]
        sx = dinv * (p_ref[0] + p_ref[1]) + (dinv * dinv) * x_ref[...]
        h = jnp.dot(sx, W1_ref[...], preferred_element_type=jnp.float32)
        h = jnp.maximum(h + b1_ref[...], 0.0)
        t = jnp.dot(h, W2_ref[...], preferred_element_type=jnp.float32)
        t_ref[...] = t
        g1_ref[...] = dinv * t

    return pl.pallas_call(
        body,
        grid=(grid,),
        in_specs=[
            pl.BlockSpec((NC, bn, d), lambda i: (0, i, 0)),
            pl.BlockSpec((bn, d), lambda i: (i, 0)),
            pl.BlockSpec((bn, 1), lambda i: (i, 0)),
            pl.BlockSpec((d, dh), lambda i: (0, 0)),
            pl.BlockSpec((1, dh), lambda i: (0, 0)),
            pl.BlockSpec((dh, do), lambda i: (0, 0)),
        ],
        out_specs=[
            pl.BlockSpec((bn, do), lambda i: (i, 0)),
            pl.BlockSpec((bn, do), lambda i: (i, 0)),
        ],
        out_shape=[
            jax.ShapeDtypeStruct((n, do), jnp.float32),
            jax.ShapeDtypeStruct((n, do), jnp.float32),
        ],
    )(p, x, dinv, W1, b1, W2)


def _tc_final(q, t, dinv, b2, bn):
    n, do = t.shape
    grid = n // bn

    def body(q_ref, t_ref, dinv_ref, b2_ref, o_ref):
        dinv = dinv_ref[...]
        o_ref[...] = (dinv * (q_ref[0] + q_ref[1])
                      + (dinv * dinv) * t_ref[...] + b2_ref[...])

    return pl.pallas_call(
        body,
        grid=(grid,),
        in_specs=[
            pl.BlockSpec((NC, bn, do), lambda i: (0, i, 0)),
            pl.BlockSpec((bn, do), lambda i: (i, 0)),
            pl.BlockSpec((bn, 1), lambda i: (i, 0)),
            pl.BlockSpec((1, do), lambda i: (0, 0)),
        ],
        out_specs=pl.BlockSpec((bn, do), lambda i: (i, 0)),
        out_shape=jax.ShapeDtypeStruct((n, do), jnp.float32),
    )(q, t, dinv, b2)


def kernel(x, edge_index, W1, b1, W2, b2):
    n, d_in = x.shape
    e = edge_index.shape[1]
    ept = e // NW
    nch = ept // CH
    bn = 1000

    src3 = edge_index[0].reshape(NW, nch, CH)
    dst3 = edge_index[1].reshape(NW, nch, CH)
    np_ = NS * (-(-n // (NS * 8)) * 8)
    zeros = jnp.zeros((np_, d_in), jnp.float32)
    ones128 = jnp.ones((CH, d_in), jnp.float32)

    degp = _sc_degree(dst3, zeros, ones128, n)
    g0, dinv = _tc_scale(degp, x, bn)
    p1 = _sc_aggregate(g0, src3, dst3, zeros, n, d_in)
    t, g1 = _tc_mid(p1, x, dinv, W1, b1.reshape(1, -1), W2, bn)
    p2 = _sc_aggregate(g1, src3, dst3, zeros, n, d_in)
    return _tc_final(p2, t, dinv, b2.reshape(1, -1), bn)


# speedup vs baseline: 29.0433x; 1.3480x over previous
import functools

import jax
import jax.numpy as jnp
from jax import lax
from jax.experimental import pallas as pl
from jax.experimental.pallas import tpu as pltpu
from jax.experimental.pallas import tpu_sc as plsc

NC = 2
NS = 16
NW = NC * NS

CH = 100


def _sc_degree(dst3, zeros, ones128, n):
    nch = dst3.shape[1]
    np_, d = zeros.shape
    rows = np_ // NS
    mesh = plsc.VectorSubcoreMesh(core_axis_name="c", subcore_axis_name="s")

    @functools.partial(
        pl.kernel,
        out_type=jax.ShapeDtypeStruct((NC, np_, d), jnp.float32),
        mesh=mesh,
        scratch_types=[
            pltpu.VMEM((nch, CH), jnp.int32),
            pltpu.VMEM((CH, d), jnp.float32),
            pltpu.VMEM_SHARED((np_, d), jnp.float32),
        ],
    )
    def k(dst_hbm, z_hbm, ones_hbm, out_hbm, didx, ones_v, acc):
        c = lax.axis_index("c")
        s = lax.axis_index("s")
        wid = s * NC + c
        pltpu.sync_copy(z_hbm.at[pl.ds(s * rows, rows)], acc.at[pl.ds(s * rows, rows)])
        pltpu.sync_copy(ones_hbm, ones_v)
        pltpu.sync_copy(dst_hbm.at[wid], didx)
        plsc.subcore_barrier()

        def body(j, carry):
            pltpu.sync_copy(ones_v, acc.at[didx.at[j]], add=True)
            return carry

        lax.fori_loop(0, nch, body, 0)
        plsc.subcore_barrier()
        pltpu.sync_copy(acc.at[pl.ds(s * rows, rows)], out_hbm.at[c, pl.ds(s * rows, rows)])

    return k(dst3, zeros, ones128)


def _sc_aggregate(g, src4, dst4, zeros, n, d):
    ng, gc = src4.shape[1], src4.shape[2]
    np_ = zeros.shape[0]
    rows = np_ // NS
    mesh = plsc.VectorSubcoreMesh(core_axis_name="c", subcore_axis_name="s")

    @functools.partial(
        pl.kernel,
        out_type=jax.ShapeDtypeStruct((NC, np_, d), jnp.float32),
        mesh=mesh,
        scratch_types=[
            pltpu.VMEM((gc, CH), jnp.int32),
            pltpu.VMEM((gc, CH), jnp.int32),
            pltpu.VMEM((CH, d), jnp.float32),
            pltpu.VMEM((CH, d), jnp.float32),
            pltpu.VMEM_SHARED((np_, d), jnp.float32),
            pltpu.SemaphoreType.DMA,
            pltpu.SemaphoreType.DMA,
        ],
    )
    def k(g_hbm, src_hbm, dst_hbm, z_hbm, out_hbm, sidx, didx, buf0, buf1, acc,
          sem0, sem1):
        c = lax.axis_index("c")
        s = lax.axis_index("s")
        wid = s * NC + c
        pltpu.sync_copy(z_hbm.at[pl.ds(s * rows, rows)], acc.at[pl.ds(s * rows, rows)])
        plsc.subcore_barrier()

        def group(gi, carry):
            pltpu.sync_copy(src_hbm.at[wid, gi], sidx)
            pltpu.sync_copy(dst_hbm.at[wid, gi], didx)
            pltpu.async_copy(g_hbm.at[sidx.at[0]], buf0, sem0)

            def body(jj, carry2):
                j0 = 2 * jj
                j1 = j0 + 1
                pltpu.async_copy(g_hbm.at[sidx.at[j1]], buf1, sem1)
                pltpu.make_async_copy(g_hbm.at[sidx.at[j0]], buf0, sem0).wait()
                pltpu.sync_copy(buf0, acc.at[didx.at[j0]], add=True)

                @pl.when(jj < gc // 2 - 1)
                def _():
                    pltpu.async_copy(g_hbm.at[sidx.at[j0 + 2]], buf0, sem0)

                pltpu.make_async_copy(g_hbm.at[sidx.at[j1]], buf1, sem1).wait()
                pltpu.sync_copy(buf1, acc.at[didx.at[j1]], add=True)
                return carry2

            lax.fori_loop(0, gc // 2, body, 0)
            return carry

        lax.fori_loop(0, ng, group, 0)
        plsc.subcore_barrier()
        pltpu.sync_copy(acc.at[pl.ds(s * rows, rows)], out_hbm.at[c, pl.ds(s * rows, rows)])

    return k(g, src4, dst4, zeros)


def _tc_scale(degp, x, bn):
    n, d = x.shape
    grid = n // bn

    def body(degp_ref, x_ref, g0_ref, dinv_ref):
        deg = degp_ref[0][:, 0:1] + degp_ref[1][:, 0:1] + 1.0
        dinv = lax.rsqrt(deg)
        dinv_ref[...] = dinv
        g0_ref[...] = x_ref[...] * dinv

    return pl.pallas_call(
        body,
        grid=(grid,),
        in_specs=[
            pl.BlockSpec((NC, bn, d), lambda i: (0, i, 0)),
            pl.BlockSpec((bn, d), lambda i: (i, 0)),
        ],
        out_specs=[
            pl.BlockSpec((bn, d), lambda i: (i, 0)),
            pl.BlockSpec((bn, 1), lambda i: (i, 0)),
        ],
        out_shape=[
            jax.ShapeDtypeStruct((n, d), jnp.float32),
            jax.ShapeDtypeStruct((n, 1), jnp.float32),
        ],
    )(degp, x)


def _tc_mid(p, x, dinv, W1, b1, W2, bn):
    n, d = x.shape
    dh = W1.shape[1]
    do = W2.shape[1]
    grid = n // bn

    def body(p_ref, x_ref, dinv_ref, W1_ref, b1_ref, W2_ref, t_ref, g1_ref):
        dinv = dinv_ref[...]
        sx = dinv * (p_ref[0] + p_ref[1]) + (dinv * dinv) * x_ref[...]
        h = jnp.dot(sx, W1_ref[...], preferred_element_type=jnp.float32)
        h = jnp.maximum(h + b1_ref[...], 0.0)
        t = jnp.dot(h, W2_ref[...], preferred_element_type=jnp.float32)
        t_ref[...] = t
        g1_ref[...] = dinv * t

    return pl.pallas_call(
        body,
        grid=(grid,),
        in_specs=[
            pl.BlockSpec((NC, bn, d), lambda i: (0, i, 0)),
            pl.BlockSpec((bn, d), lambda i: (i, 0)),
            pl.BlockSpec((bn, 1), lambda i: (i, 0)),
            pl.BlockSpec((d, dh), lambda i: (0, 0)),
            pl.BlockSpec((1, dh), lambda i: (0, 0)),
            pl.BlockSpec((dh, do), lambda i: (0, 0)),
        ],
        out_specs=[
            pl.BlockSpec((bn, do), lambda i: (i, 0)),
            pl.BlockSpec((bn, do), lambda i: (i, 0)),
        ],
        out_shape=[
            jax.ShapeDtypeStruct((n, do), jnp.float32),
            jax.ShapeDtypeStruct((n, do), jnp.float32),
        ],
    )(p, x, dinv, W1, b1, W2)


def _tc_final(q, t, dinv, b2, bn):
    n, do = t.shape
    grid = n // bn

    def body(q_ref, t_ref, dinv_ref, b2_ref, o_ref):
        dinv = dinv_ref[...]
        o_ref[...] = (dinv * (q_ref[0] + q_ref[1])
                      + (dinv * dinv) * t_ref[...] + b2_ref[...])

    return pl.pallas_call(
        body,
        grid=(grid,),
        in_specs=[
            pl.BlockSpec((NC, bn, do), lambda i: (0, i, 0)),
            pl.BlockSpec((bn, do), lambda i: (i, 0)),
            pl.BlockSpec((bn, 1), lambda i: (i, 0)),
            pl.BlockSpec((1, do), lambda i: (0, 0)),
        ],
        out_specs=pl.BlockSpec((bn, do), lambda i: (i, 0)),
        out_shape=jax.ShapeDtypeStruct((n, do), jnp.float32),
    )(q, t, dinv, b2)


def kernel(x, edge_index, W1, b1, W2, b2):
    n, d_in = x.shape
    e = edge_index.shape[1]
    ept = e // NW
    nch = ept // CH
    bn = 1000

    gc = 20
    ng = nch // gc
    src4 = edge_index[0].reshape(NW, ng, gc, CH)
    dst4 = edge_index[1].reshape(NW, ng, gc, CH)
    dst3 = edge_index[1].reshape(NW, nch, CH)
    np_ = NS * (-(-n // (NS * 8)) * 8)
    zeros = jnp.zeros((np_, d_in), jnp.float32)
    ones128 = jnp.ones((CH, d_in), jnp.float32)

    degp = _sc_degree(dst3, zeros, ones128, n)
    g0, dinv = _tc_scale(degp, x, bn)
    p1 = _sc_aggregate(g0, src4, dst4, zeros, n, d_in)
    t, g1 = _tc_mid(p1, x, dinv, W1, b1.reshape(1, -1), W2, bn)
    p2 = _sc_aggregate(g1, src4, dst4, zeros, n, d_in)
    return _tc_final(p2, t, dinv, b2.reshape(1, -1), bn)


# speedup vs baseline: 29.9874x; 1.0325x over previous
import functools

import jax
import jax.numpy as jnp
from jax import lax
from jax.experimental import pallas as pl
from jax.experimental.pallas import tpu as pltpu
from jax.experimental.pallas import tpu_sc as plsc

NC = 2
NS = 16
NW = NC * NS

CH = 125


def _sc_degree(dst3, zeros, ones128, n):
    nch = dst3.shape[1]
    np_, d = zeros.shape
    rows = np_ // NS
    mesh = plsc.VectorSubcoreMesh(core_axis_name="c", subcore_axis_name="s")

    @functools.partial(
        pl.kernel,
        out_type=jax.ShapeDtypeStruct((NC, np_, d), jnp.float32),
        mesh=mesh,
        scratch_types=[
            pltpu.VMEM((nch, CH), jnp.int32),
            pltpu.VMEM((CH, d), jnp.float32),
            pltpu.VMEM_SHARED((np_, d), jnp.float32),
        ],
    )
    def k(dst_hbm, z_hbm, ones_hbm, out_hbm, didx, ones_v, acc):
        c = lax.axis_index("c")
        s = lax.axis_index("s")
        wid = s * NC + c
        pltpu.sync_copy(z_hbm.at[pl.ds(s * rows, rows)], acc.at[pl.ds(s * rows, rows)])
        pltpu.sync_copy(ones_hbm, ones_v)
        pltpu.sync_copy(dst_hbm.at[wid], didx)
        plsc.subcore_barrier()

        def body(j, carry):
            pltpu.sync_copy(ones_v, acc.at[didx.at[j]], add=True)
            return carry

        lax.fori_loop(0, nch, body, 0)
        plsc.subcore_barrier()
        pltpu.sync_copy(acc.at[pl.ds(s * rows, rows)], out_hbm.at[c, pl.ds(s * rows, rows)])

    return k(dst3, zeros, ones128)


def _sc_aggregate(g, src4, dst4, zeros, n, d):
    ng, gc = src4.shape[1], src4.shape[2]
    np_ = zeros.shape[0]
    rows = np_ // NS
    mesh = plsc.VectorSubcoreMesh(core_axis_name="c", subcore_axis_name="s")

    @functools.partial(
        pl.kernel,
        out_type=jax.ShapeDtypeStruct((NC, np_, d), jnp.float32),
        mesh=mesh,
        scratch_types=[
            pltpu.VMEM((gc, CH), jnp.int32),
            pltpu.VMEM((gc, CH), jnp.int32),
            pltpu.VMEM((CH, d), jnp.float32),
            pltpu.VMEM((CH, d), jnp.float32),
            pltpu.VMEM_SHARED((np_, d), jnp.float32),
            pltpu.SemaphoreType.DMA,
            pltpu.SemaphoreType.DMA,
        ],
    )
    def k(g_hbm, src_hbm, dst_hbm, z_hbm, out_hbm, sidx, didx, buf0, buf1, acc,
          sem0, sem1):
        c = lax.axis_index("c")
        s = lax.axis_index("s")
        wid = s * NC + c
        pltpu.sync_copy(z_hbm.at[pl.ds(s * rows, rows)], acc.at[pl.ds(s * rows, rows)])
        plsc.subcore_barrier()

        def group(gi, carry):
            pltpu.sync_copy(src_hbm.at[wid, gi], sidx)
            pltpu.sync_copy(dst_hbm.at[wid, gi], didx)
            pltpu.async_copy(g_hbm.at[sidx.at[0]], buf0, sem0)

            def body(jj, carry2):
                j0 = 2 * jj
                j1 = j0 + 1
                pltpu.async_copy(g_hbm.at[sidx.at[j1]], buf1, sem1)
                pltpu.make_async_copy(g_hbm.at[sidx.at[j0]], buf0, sem0).wait()
                pltpu.sync_copy(buf0, acc.at[didx.at[j0]], add=True)

                @pl.when(jj < gc // 2 - 1)
                def _():
                    pltpu.async_copy(g_hbm.at[sidx.at[j0 + 2]], buf0, sem0)

                pltpu.make_async_copy(g_hbm.at[sidx.at[j1]], buf1, sem1).wait()
                pltpu.sync_copy(buf1, acc.at[didx.at[j1]], add=True)
                return carry2

            lax.fori_loop(0, gc // 2, body, 0)
            return carry

        lax.fori_loop(0, ng, group, 0)
        plsc.subcore_barrier()
        pltpu.sync_copy(acc.at[pl.ds(s * rows, rows)], out_hbm.at[c, pl.ds(s * rows, rows)])

    return k(g, src4, dst4, zeros)


def _tc_scale(degp, x, bn):
    n, d = x.shape
    grid = n // bn

    def body(degp_ref, x_ref, g0_ref, dinv_ref):
        deg = degp_ref[0][:, 0:1] + degp_ref[1][:, 0:1] + 1.0
        dinv = lax.rsqrt(deg)
        dinv_ref[...] = dinv
        g0_ref[...] = x_ref[...] * dinv

    return pl.pallas_call(
        body,
        grid=(grid,),
        in_specs=[
            pl.BlockSpec((NC, bn, d), lambda i: (0, i, 0)),
            pl.BlockSpec((bn, d), lambda i: (i, 0)),
        ],
        out_specs=[
            pl.BlockSpec((bn, d), lambda i: (i, 0)),
            pl.BlockSpec((bn, 1), lambda i: (i, 0)),
        ],
        out_shape=[
            jax.ShapeDtypeStruct((n, d), jnp.float32),
            jax.ShapeDtypeStruct((n, 1), jnp.float32),
        ],
    )(degp, x)


def _tc_mid(p, x, dinv, W1, b1, W2, bn):
    n, d = x.shape
    dh = W1.shape[1]
    do = W2.shape[1]
    grid = n // bn

    def body(p_ref, x_ref, dinv_ref, W1_ref, b1_ref, W2_ref, t_ref, g1_ref):
        dinv = dinv_ref[...]
        sx = dinv * (p_ref[0] + p_ref[1]) + (dinv * dinv) * x_ref[...]
        h = jnp.dot(sx, W1_ref[...], preferred_element_type=jnp.float32)
        h = jnp.maximum(h + b1_ref[...], 0.0)
        t = jnp.dot(h, W2_ref[...], preferred_element_type=jnp.float32)
        t_ref[...] = t
        g1_ref[...] = dinv * t

    return pl.pallas_call(
        body,
        grid=(grid,),
        in_specs=[
            pl.BlockSpec((NC, bn, d), lambda i: (0, i, 0)),
            pl.BlockSpec((bn, d), lambda i: (i, 0)),
            pl.BlockSpec((bn, 1), lambda i: (i, 0)),
            pl.BlockSpec((d, dh), lambda i: (0, 0)),
            pl.BlockSpec((1, dh), lambda i: (0, 0)),
            pl.BlockSpec((dh, do), lambda i: (0, 0)),
        ],
        out_specs=[
            pl.BlockSpec((bn, do), lambda i: (i, 0)),
            pl.BlockSpec((bn, do), lambda i: (i, 0)),
        ],
        out_shape=[
            jax.ShapeDtypeStruct((n, do), jnp.float32),
            jax.ShapeDtypeStruct((n, do), jnp.float32),
        ],
    )(p, x, dinv, W1, b1, W2)


def _tc_final(q, t, dinv, b2, bn):
    n, do = t.shape
    grid = n // bn

    def body(q_ref, t_ref, dinv_ref, b2_ref, o_ref):
        dinv = dinv_ref[...]
        o_ref[...] = (dinv * (q_ref[0] + q_ref[1])
                      + (dinv * dinv) * t_ref[...] + b2_ref[...])

    return pl.pallas_call(
        body,
        grid=(grid,),
        in_specs=[
            pl.BlockSpec((NC, bn, do), lambda i: (0, i, 0)),
            pl.BlockSpec((bn, do), lambda i: (i, 0)),
            pl.BlockSpec((bn, 1), lambda i: (i, 0)),
            pl.BlockSpec((1, do), lambda i: (0, 0)),
        ],
        out_specs=pl.BlockSpec((bn, do), lambda i: (i, 0)),
        out_shape=jax.ShapeDtypeStruct((n, do), jnp.float32),
    )(q, t, dinv, b2)


def kernel(x, edge_index, W1, b1, W2, b2):
    n, d_in = x.shape
    e = edge_index.shape[1]
    ept = e // NW
    nch = ept // CH
    bn = 1000

    gc = 20
    ng = nch // gc
    src4 = edge_index[0].reshape(NW, ng, gc, CH)
    dst4 = edge_index[1].reshape(NW, ng, gc, CH)
    dst3 = edge_index[1].reshape(NW, nch, CH)
    np_ = NS * (-(-n // (NS * 8)) * 8)
    zeros = jnp.zeros((np_, d_in), jnp.float32)
    ones128 = jnp.ones((CH, d_in), jnp.float32)

    degp = _sc_degree(dst3, zeros, ones128, n)
    g0, dinv = _tc_scale(degp, x, bn)
    p1 = _sc_aggregate(g0, src4, dst4, zeros, n, d_in)
    t, g1 = _tc_mid(p1, x, dinv, W1, b1.reshape(1, -1), W2, bn)
    p2 = _sc_aggregate(g1, src4, dst4, zeros, n, d_in)
    return _tc_final(p2, t, dinv, b2.reshape(1, -1), bn)


# speedup vs baseline: 30.8063x; 1.0273x over previous
import functools

import jax
import jax.numpy as jnp
from jax import lax
from jax.experimental import pallas as pl
from jax.experimental.pallas import tpu as pltpu
from jax.experimental.pallas import tpu_sc as plsc

NC = 2
NS = 16
NW = NC * NS

CH = 100
CHD = 125


def _sc_degree(dst3, zeros, ones128, n):
    nch, chd = dst3.shape[1], dst3.shape[2]
    np_, d = zeros.shape
    rows = 1000
    mesh = plsc.VectorSubcoreMesh(core_axis_name="c", subcore_axis_name="s")

    @functools.partial(
        pl.kernel,
        out_type=jax.ShapeDtypeStruct((NC, np_, d), jnp.float32),
        mesh=mesh,
        scratch_types=[
            pltpu.VMEM((nch, chd), jnp.int32),
            pltpu.VMEM((chd, d), jnp.float32),
            pltpu.VMEM_SHARED((np_, d), jnp.float32),
            pltpu.SemaphoreType.DMA,
        ],
    )
    def k(dst_hbm, z_hbm, ones_hbm, out_hbm, didx, ones_v, acc, sem):
        c = lax.axis_index("c")
        s = lax.axis_index("s")
        wid = s * NC + c

        @pl.when(s < 10)
        def _():
            pltpu.sync_copy(z_hbm.at[pl.ds(s * rows, rows)], acc.at[pl.ds(s * rows, rows)])

        pltpu.sync_copy(ones_hbm, ones_v)
        pltpu.sync_copy(dst_hbm.at[wid], didx)
        plsc.subcore_barrier()

        pltpu.async_copy(ones_v, acc.at[didx.at[0]], sem, add=True)
        pltpu.async_copy(ones_v, acc.at[didx.at[1]], sem, add=True)

        def body(j, carry):
            @pl.when(j + 2 < nch)
            def _():
                pltpu.async_copy(ones_v, acc.at[didx.at[j + 2]], sem, add=True)

            pltpu.make_async_copy(ones_v, acc.at[didx.at[0]], sem).wait()
            return carry

        lax.fori_loop(0, nch, body, 0)
        plsc.subcore_barrier()

        @pl.when(s < 10)
        def _():
            pltpu.sync_copy(acc.at[pl.ds(s * rows, rows)], out_hbm.at[c, pl.ds(s * rows, rows)])

    return k(dst3, zeros, ones128)


def _sc_aggregate(g, packed, zeros, n, d):
    nch = packed.shape[1]
    np_ = zeros.shape[0]
    rows = 1000
    mesh = plsc.VectorSubcoreMesh(core_axis_name="c", subcore_axis_name="s")

    @functools.partial(
        pl.kernel,
        out_type=jax.ShapeDtypeStruct((NC, np_, d), jnp.float32),
        mesh=mesh,
        scratch_types=[
            pltpu.VMEM((3, 2, CH), jnp.int32),
            pltpu.VMEM((CH, d), jnp.float32),
            pltpu.VMEM((CH, d), jnp.float32),
            pltpu.VMEM((CH, d), jnp.float32),
            pltpu.VMEM_SHARED((np_, d), jnp.float32),
            pltpu.SemaphoreType.DMA,
            pltpu.SemaphoreType.DMA,
            pltpu.SemaphoreType.DMA,
            pltpu.SemaphoreType.DMA,
            pltpu.SemaphoreType.DMA,
            pltpu.SemaphoreType.DMA,
        ],
    )
    def k(g_hbm, idx_hbm, z_hbm, out_hbm, islot, buf0, buf1, buf2, acc,
          sg0, sg1, sg2, si0, si1, si2):
        c = lax.axis_index("c")
        s = lax.axis_index("s")
        wid = s * NC + c
        bufs = (buf0, buf1, buf2)
        sgs = (sg0, sg1, sg2)
        sis = (si0, si1, si2)

        @pl.when(s < 10)
        def _():
            pltpu.sync_copy(z_hbm.at[pl.ds(s * rows, rows)], acc.at[pl.ds(s * rows, rows)])

        plsc.subcore_barrier()

        pltpu.sync_copy(idx_hbm.at[wid, 0], islot.at[0])
        pltpu.sync_copy(idx_hbm.at[wid, 1], islot.at[1])
        pltpu.async_copy(idx_hbm.at[wid, 2], islot.at[2], sis[2])
        pltpu.async_copy(g_hbm.at[islot.at[0, 0]], buf0, sgs[0])
        pltpu.async_copy(g_hbm.at[islot.at[1, 0]], buf1, sgs[1])

        def chunk(j, a):
            an = (a + 2) % 3
            pltpu.make_async_copy(g_hbm.at[islot.at[a, 0]], bufs[a], sgs[a]).wait()
            pltpu.sync_copy(bufs[a], acc.at[islot.at[a, 1]], add=True)

            @pl.when(j + 3 < nch)
            def _():
                pltpu.async_copy(idx_hbm.at[wid, j + 3], islot.at[a], sis[a])

            @pl.when(j + 2 < nch)
            def _():
                pltpu.make_async_copy(idx_hbm.at[wid, 0], islot.at[an], sis[an]).wait()
                pltpu.async_copy(g_hbm.at[islot.at[an, 0]], bufs[an], sgs[an])

        def triple(t, carry):
            j0 = 3 * t
            chunk(j0, 0)
            chunk(j0 + 1, 1)
            chunk(j0 + 2, 2)
            return carry

        lax.fori_loop(0, nch // 3, triple, 0)
        for j in range(3 * (nch // 3), nch):
            a = j % 3
            pltpu.make_async_copy(g_hbm.at[islot.at[a, 0]], bufs[a], sgs[a]).wait()
            pltpu.sync_copy(bufs[a], acc.at[islot.at[a, 1]], add=True)

        plsc.subcore_barrier()

        @pl.when(s < 10)
        def _():
            pltpu.sync_copy(acc.at[pl.ds(s * rows, rows)], out_hbm.at[c, pl.ds(s * rows, rows)])

    return k(g, packed, zeros)


def _tc_scale(degp, x, bn):
    n, d = x.shape
    grid = n // bn

    def body(degp_ref, x_ref, g0_ref, dinv_ref):
        deg = degp_ref[0][:, 0:1] + degp_ref[1][:, 0:1] + 1.0
        dinv = lax.rsqrt(deg)
        dinv_ref[...] = dinv
        g0_ref[...] = x_ref[...] * dinv

    return pl.pallas_call(
        body,
        grid=(grid,),
        in_specs=[
            pl.BlockSpec((NC, bn, d), lambda i: (0, i, 0)),
            pl.BlockSpec((bn, d), lambda i: (i, 0)),
        ],
        out_specs=[
            pl.BlockSpec((bn, d), lambda i: (i, 0)),
            pl.BlockSpec((bn, 1), lambda i: (i, 0)),
        ],
        out_shape=[
            jax.ShapeDtypeStruct((n, d), jnp.float32),
            jax.ShapeDtypeStruct((n, 1), jnp.float32),
        ],
    )(degp, x)


def _tc_mid(p, x, dinv, W1, b1, W2, bn):
    n, d = x.shape
    dh = W1.shape[1]
    do = W2.shape[1]
    grid = n // bn

    def body(p_ref, x_ref, dinv_ref, W1_ref, b1_ref, W2_ref, t_ref, g1_ref):
        dinv = dinv_ref[...]
        sx = dinv * (p_ref[0] + p_ref[1]) + (dinv * dinv) * x_ref[...]
        h = jnp.dot(sx, W1_ref[...], preferred_element_type=jnp.float32)
        h = jnp.maximum(h + b1_ref[...], 0.0)
        t = jnp.dot(h, W2_ref[...], preferred_element_type=jnp.float32)
        t_ref[...] = t
        g1_ref[...] = dinv * t

    return pl.pallas_call(
        body,
        grid=(grid,),
        in_specs=[
            pl.BlockSpec((NC, bn, d), lambda i: (0, i, 0)),
            pl.BlockSpec((bn, d), lambda i: (i, 0)),
            pl.BlockSpec((bn, 1), lambda i: (i, 0)),
            pl.BlockSpec((d, dh), lambda i: (0, 0)),
            pl.BlockSpec((1, dh), lambda i: (0, 0)),
            pl.BlockSpec((dh, do), lambda i: (0, 0)),
        ],
        out_specs=[
            pl.BlockSpec((bn, do), lambda i: (i, 0)),
            pl.BlockSpec((bn, do), lambda i: (i, 0)),
        ],
        out_shape=[
            jax.ShapeDtypeStruct((n, do), jnp.float32),
            jax.ShapeDtypeStruct((n, do), jnp.float32),
        ],
    )(p, x, dinv, W1, b1, W2)


def _tc_final(q, t, dinv, b2, bn):
    n, do = t.shape
    grid = n // bn

    def body(q_ref, t_ref, dinv_ref, b2_ref, o_ref):
        dinv = dinv_ref[...]
        o_ref[...] = (dinv * (q_ref[0] + q_ref[1])
                      + (dinv * dinv) * t_ref[...] + b2_ref[...])

    return pl.pallas_call(
        body,
        grid=(grid,),
        in_specs=[
            pl.BlockSpec((NC, bn, do), lambda i: (0, i, 0)),
            pl.BlockSpec((bn, do), lambda i: (i, 0)),
            pl.BlockSpec((bn, 1), lambda i: (i, 0)),
            pl.BlockSpec((1, do), lambda i: (0, 0)),
        ],
        out_specs=pl.BlockSpec((bn, do), lambda i: (i, 0)),
        out_shape=jax.ShapeDtypeStruct((n, do), jnp.float32),
    )(q, t, dinv, b2)


def kernel(x, edge_index, W1, b1, W2, b2):
    n, d_in = x.shape
    e = edge_index.shape[1]
    ept = e // NW
    bn = 1000

    ncha = ept // CH
    nchd = ept // CHD
    packed = edge_index.reshape(2, NW, ncha, CH).transpose(1, 2, 0, 3)
    dst3 = edge_index[1].reshape(NW, nchd, CHD)
    zeros = jnp.zeros((n, d_in), jnp.float32)
    ones128 = jnp.ones((CHD, d_in), jnp.float32)

    degp = _sc_degree(dst3, zeros, ones128, n)
    g0, dinv = _tc_scale(degp, x, bn)
    p1 = _sc_aggregate(g0, packed, zeros, n, d_in)
    t, g1 = _tc_mid(p1, x, dinv, W1, b1.reshape(1, -1), W2, bn)
    p2 = _sc_aggregate(g1, packed, zeros, n, d_in)
    return _tc_final(p2, t, dinv, b2.reshape(1, -1), bn)


# speedup vs baseline: 31.7353x; 1.0302x over previous
import functools

import jax
import jax.numpy as jnp
from jax import lax
from jax.experimental import pallas as pl
from jax.experimental.pallas import tpu as pltpu
from jax.experimental.pallas import tpu_sc as plsc

NC = 2
NS = 16
NW = NC * NS

CH = 125
CHD = 125


def _sc_degree(dst3, zeros, ones128, n):
    nch, chd = dst3.shape[1], dst3.shape[2]
    np_, d = zeros.shape
    rows = 1000
    mesh = plsc.VectorSubcoreMesh(core_axis_name="c", subcore_axis_name="s")

    @functools.partial(
        pl.kernel,
        out_type=jax.ShapeDtypeStruct((NC, np_, d), jnp.float32),
        mesh=mesh,
        scratch_types=[
            pltpu.VMEM((nch, chd), jnp.int32),
            pltpu.VMEM((chd, d), jnp.float32),
            pltpu.VMEM_SHARED((np_, d), jnp.float32),
            pltpu.SemaphoreType.DMA,
        ],
    )
    def k(dst_hbm, z_hbm, ones_hbm, out_hbm, didx, ones_v, acc, sem):
        c = lax.axis_index("c")
        s = lax.axis_index("s")
        wid = s * NC + c

        @pl.when(s < 10)
        def _():
            pltpu.sync_copy(z_hbm.at[pl.ds(s * rows, rows)], acc.at[pl.ds(s * rows, rows)])

        pltpu.sync_copy(ones_hbm, ones_v)
        pltpu.sync_copy(dst_hbm.at[wid], didx)
        plsc.subcore_barrier()

        pltpu.async_copy(ones_v, acc.at[didx.at[0]], sem, add=True)
        pltpu.async_copy(ones_v, acc.at[didx.at[1]], sem, add=True)

        def body(j, carry):
            @pl.when(j + 2 < nch)
            def _():
                pltpu.async_copy(ones_v, acc.at[didx.at[j + 2]], sem, add=True)

            pltpu.make_async_copy(ones_v, acc.at[didx.at[0]], sem).wait()
            return carry

        lax.fori_loop(0, nch, body, 0)
        plsc.subcore_barrier()

        @pl.when(s < 10)
        def _():
            pltpu.sync_copy(acc.at[pl.ds(s * rows, rows)], out_hbm.at[c, pl.ds(s * rows, rows)])

    return k(dst3, zeros, ones128)


def _sc_aggregate(g, packed, zeros, n, d):
    nch = packed.shape[1]
    np_ = zeros.shape[0]
    rows = 1000
    mesh = plsc.VectorSubcoreMesh(core_axis_name="c", subcore_axis_name="s")

    @functools.partial(
        pl.kernel,
        out_type=jax.ShapeDtypeStruct((NC, np_, d), jnp.float32),
        mesh=mesh,
        scratch_types=[
            pltpu.VMEM((3, 2, CH), jnp.int32),
            pltpu.VMEM((CH, d), jnp.float32),
            pltpu.VMEM((CH, d), jnp.float32),
            pltpu.VMEM((CH, d), jnp.float32),
            pltpu.VMEM_SHARED((np_, d), jnp.float32),
            pltpu.SemaphoreType.DMA,
            pltpu.SemaphoreType.DMA,
            pltpu.SemaphoreType.DMA,
            pltpu.SemaphoreType.DMA,
            pltpu.SemaphoreType.DMA,
            pltpu.SemaphoreType.DMA,
        ],
    )
    def k(g_hbm, idx_hbm, z_hbm, out_hbm, islot, buf0, buf1, buf2, acc,
          sg0, sg1, sg2, si0, si1, si2):
        c = lax.axis_index("c")
        s = lax.axis_index("s")
        wid = s * NC + c
        bufs = (buf0, buf1, buf2)
        sgs = (sg0, sg1, sg2)
        sis = (si0, si1, si2)

        @pl.when(s < 10)
        def _():
            pltpu.sync_copy(z_hbm.at[pl.ds(s * rows, rows)], acc.at[pl.ds(s * rows, rows)])

        plsc.subcore_barrier()

        pltpu.sync_copy(idx_hbm.at[wid, 0], islot.at[0])
        pltpu.sync_copy(idx_hbm.at[wid, 1], islot.at[1])
        pltpu.async_copy(idx_hbm.at[wid, 2], islot.at[2], sis[2])
        pltpu.async_copy(g_hbm.at[islot.at[0, 0]], buf0, sgs[0])
        pltpu.async_copy(g_hbm.at[islot.at[1, 0]], buf1, sgs[1])

        def chunk(j, a):
            an = (a + 2) % 3
            pltpu.make_async_copy(g_hbm.at[islot.at[a, 0]], bufs[a], sgs[a]).wait()
            pltpu.sync_copy(bufs[a], acc.at[islot.at[a, 1]], add=True)

            @pl.when(j + 3 < nch)
            def _():
                pltpu.async_copy(idx_hbm.at[wid, j + 3], islot.at[a], sis[a])

            @pl.when(j + 2 < nch)
            def _():
                pltpu.make_async_copy(idx_hbm.at[wid, 0], islot.at[an], sis[an]).wait()
                pltpu.async_copy(g_hbm.at[islot.at[an, 0]], bufs[an], sgs[an])

        def triple(t, carry):
            j0 = 3 * t
            chunk(j0, 0)
            chunk(j0 + 1, 1)
            chunk(j0 + 2, 2)
            return carry

        lax.fori_loop(0, nch // 3, triple, 0)
        for j in range(3 * (nch // 3), nch):
            a = j % 3
            pltpu.make_async_copy(g_hbm.at[islot.at[a, 0]], bufs[a], sgs[a]).wait()
            pltpu.sync_copy(bufs[a], acc.at[islot.at[a, 1]], add=True)

        plsc.subcore_barrier()

        @pl.when(s < 10)
        def _():
            pltpu.sync_copy(acc.at[pl.ds(s * rows, rows)], out_hbm.at[c, pl.ds(s * rows, rows)])

    return k(g, packed, zeros)


def _tc_scale(degp, x, bn):
    n, d = x.shape
    grid = n // bn

    def body(degp_ref, x_ref, g0_ref, dinv_ref):
        deg = degp_ref[0][:, 0:1] + degp_ref[1][:, 0:1] + 1.0
        dinv = lax.rsqrt(deg)
        dinv_ref[...] = dinv
        g0_ref[...] = x_ref[...] * dinv

    return pl.pallas_call(
        body,
        grid=(grid,),
        in_specs=[
            pl.BlockSpec((NC, bn, d), lambda i: (0, i, 0)),
            pl.BlockSpec((bn, d), lambda i: (i, 0)),
        ],
        out_specs=[
            pl.BlockSpec((bn, d), lambda i: (i, 0)),
            pl.BlockSpec((bn, 1), lambda i: (i, 0)),
        ],
        out_shape=[
            jax.ShapeDtypeStruct((n, d), jnp.float32),
            jax.ShapeDtypeStruct((n, 1), jnp.float32),
        ],
    )(degp, x)


def _tc_mid(p, x, dinv, W1, b1, W2, bn):
    n, d = x.shape
    dh = W1.shape[1]
    do = W2.shape[1]
    grid = n // bn

    def body(p_ref, x_ref, dinv_ref, W1_ref, b1_ref, W2_ref, t_ref, g1_ref):
        dinv = dinv_ref[...]
        sx = dinv * (p_ref[0] + p_ref[1]) + (dinv * dinv) * x_ref[...]
        h = jnp.dot(sx, W1_ref[...], preferred_element_type=jnp.float32)
        h = jnp.maximum(h + b1_ref[...], 0.0)
        t = jnp.dot(h, W2_ref[...], preferred_element_type=jnp.float32)
        t_ref[...] = t
        g1_ref[...] = dinv * t

    return pl.pallas_call(
        body,
        grid=(grid,),
        in_specs=[
            pl.BlockSpec((NC, bn, d), lambda i: (0, i, 0)),
            pl.BlockSpec((bn, d), lambda i: (i, 0)),
            pl.BlockSpec((bn, 1), lambda i: (i, 0)),
            pl.BlockSpec((d, dh), lambda i: (0, 0)),
            pl.BlockSpec((1, dh), lambda i: (0, 0)),
            pl.BlockSpec((dh, do), lambda i: (0, 0)),
        ],
        out_specs=[
            pl.BlockSpec((bn, do), lambda i: (i, 0)),
            pl.BlockSpec((bn, do), lambda i: (i, 0)),
        ],
        out_shape=[
            jax.ShapeDtypeStruct((n, do), jnp.float32),
            jax.ShapeDtypeStruct((n, do), jnp.float32),
        ],
    )(p, x, dinv, W1, b1, W2)


def _tc_final(q, t, dinv, b2, bn):
    n, do = t.shape
    grid = n // bn

    def body(q_ref, t_ref, dinv_ref, b2_ref, o_ref):
        dinv = dinv_ref[...]
        o_ref[...] = (dinv * (q_ref[0] + q_ref[1])
                      + (dinv * dinv) * t_ref[...] + b2_ref[...])

    return pl.pallas_call(
        body,
        grid=(grid,),
        in_specs=[
            pl.BlockSpec((NC, bn, do), lambda i: (0, i, 0)),
            pl.BlockSpec((bn, do), lambda i: (i, 0)),
            pl.BlockSpec((bn, 1), lambda i: (i, 0)),
            pl.BlockSpec((1, do), lambda i: (0, 0)),
        ],
        out_specs=pl.BlockSpec((bn, do), lambda i: (i, 0)),
        out_shape=jax.ShapeDtypeStruct((n, do), jnp.float32),
    )(q, t, dinv, b2)


def kernel(x, edge_index, W1, b1, W2, b2):
    n, d_in = x.shape
    e = edge_index.shape[1]
    ept = e // NW
    bn = 1000

    ncha = ept // CH
    nchd = ept // CHD
    packed = edge_index.reshape(2, NW, ncha, CH).transpose(1, 2, 0, 3)
    dst3 = edge_index[1].reshape(NW, nchd, CHD)
    zeros = jnp.zeros((n, d_in), jnp.float32)
    ones128 = jnp.ones((CHD, d_in), jnp.float32)

    degp = _sc_degree(dst3, zeros, ones128, n)
    g0, dinv = _tc_scale(degp, x, bn)
    p1 = _sc_aggregate(g0, packed, zeros, n, d_in)
    t, g1 = _tc_mid(p1, x, dinv, W1, b1.reshape(1, -1), W2, bn)
    p2 = _sc_aggregate(g1, packed, zeros, n, d_in)
    return _tc_final(p2, t, dinv, b2.reshape(1, -1), bn)
